# baseline (speedup 1.0000x reference)
import jax
import jax.numpy as jnp
from jax.experimental import pallas as pl
from jax.experimental.pallas import tpu as pltpu

_B, _C, _L, _E, _D = 64, 32, 2048, 8, 768
_TAU = 1.0
_BB = 16
_NG = _B // _BB


def _gelu(v):
    return v * (0.5 + 0.5 * jax.lax.erf(v * 0.7071067811865476))


def _body(x_ref, w_ref, p_ref, bnw_ref, bnb_ref, bnm_ref, bnv_ref,
          fc1w_ref, fc1b_ref, fc2w_ref, fc2b_ref, gum_ref, expw_ref,
          expb_ref, out_ref, feat_s, pooled_s, xs_s):
    i = pl.program_id(0)
    W = w_ref[...]
    P = p_ref[...]

    @pl.when(i == 0)
    def _init():
        xs_s[_C * 3:, :] = jnp.concatenate(
            [jnp.ones((1, _BB * _L), jnp.float32),
             jnp.zeros((7, _BB * _L), jnp.float32)], axis=0)

    for j in range(_BB):
        X = x_ref[j]
        z = jnp.zeros((_C, 1), jnp.float32)
        Xl = jnp.concatenate([z, X[:, :-1]], axis=1)
        Xr = jnp.concatenate([X[:, 1:], z], axis=1)
        xs_s[:_C * 3, pl.ds(j * _L, _L)] = jnp.concatenate([Xl, X, Xr], axis=0)
    H2 = jax.lax.dot_general(W, xs_s[...], (((1,), (0,)), ((), ())),
                             preferred_element_type=jnp.float32)
    G = H2 * (1.0 + jax.lax.erf(H2))
    for j in range(_BB):
        feat_s[i * _BB + j] = jnp.dot(G[:, j * _L:(j + 1) * _L], P,
                                      preferred_element_type=jnp.float32)
    pooled_s[pl.ds(i * _BB, _BB)] = jnp.mean(x_ref[...], axis=2)

    @pl.when(i == _NG - 1)
    def _stage2():
        f = feat_s[...].reshape(_B, 64 * 16)
        f = (f - bnm_ref[...]) * jax.lax.rsqrt(bnv_ref[...] + 1e-5) \
            * bnw_ref[...] + bnb_ref[...]
        h1 = _gelu(jnp.dot(f, fc1w_ref[...],
                           preferred_element_type=jnp.float32)
                   + fc1b_ref[...])
        logits = jnp.dot(h1, fc2w_ref[...],
                         preferred_element_type=jnp.float32) + fc2b_ref[...]
        zz = (logits + gum_ref[...]) / _TAU
        zz = zz - jnp.max(zz, axis=1, keepdims=True)
        ez = jnp.exp(zz)
        r = ez / jnp.sum(ez, axis=1, keepdims=True)

        col = jax.lax.broadcasted_iota(jnp.int32, (_B, _E), 1)
        m1 = jnp.max(r, axis=1, keepdims=True)
        i1 = jnp.min(jnp.where(r == m1, col, _E), axis=1, keepdims=True)
        rm = jnp.where(col == i1, -jnp.inf, r)
        m2 = jnp.max(rm, axis=1, keepdims=True)
        i2 = jnp.min(jnp.where(rm == m2, col, _E), axis=1, keepdims=True)
        s = m1 + m2 + 1e-8
        wfull = (jnp.where(col == i1, m1 / s, 0.0)
                 + jnp.where(col == i2, m2 / s, 0.0))

        ao = jnp.tanh(jnp.dot(pooled_s[...], expw_ref[...],
                              preferred_element_type=jnp.float32)
                      + expb_ref[...])
        acc = jnp.zeros((_B, _D), jnp.float32)
        for e in range(_E):
            acc = acc + wfull[:, e:e + 1] * ao[:, e * _D:(e + 1) * _D]
        out_ref[...] = acc


def kernel(x, conv_w, conv_b, bn_w, bn_b, bn_mean, bn_var,
           fc1_w, fc1_b, fc2_w, fc2_b, gumbel, exp_w, exp_b):
    rs2 = 0.7071067811865476
    w104 = jnp.concatenate(
        [jnp.transpose(conv_w, (0, 2, 1)).reshape(64, 96) * rs2,
         conv_b.reshape(64, 1) * rs2,
         jnp.zeros((64, 7), jnp.float32)], axis=1)
    expw2 = jnp.transpose(exp_w, (1, 0, 2)).reshape(_C, _E * _D)
    expb2 = exp_b.reshape(1, _E * _D)
    pool_mat = (jnp.arange(_L)[:, None] // 128
                == jnp.arange(16)[None, :]).astype(jnp.float32) * (rs2 / 128.0)

    cst = lambda *dims: pl.BlockSpec(dims, lambda i: (0,) * len(dims))
    out = pl.pallas_call(
        _body,
        grid=(_NG,),
        in_specs=[
            pl.BlockSpec((_BB, _C, _L), lambda i: (i, 0, 0)),
            cst(64, 104), cst(_L, 16),
            cst(1, 1024), cst(1, 1024), cst(1, 1024), cst(1, 1024),
            cst(1024, 128), cst(1, 128), cst(128, _E), cst(1, _E),
            cst(_B, _E), cst(_C, _E * _D), cst(1, _E * _D),
        ],
        out_specs=pl.BlockSpec((_B, _D), lambda i: (0, 0)),
        out_shape=jax.ShapeDtypeStruct((_B, _D), jnp.float32),
        scratch_shapes=[
            pltpu.VMEM((_B, 64, 16), jnp.float32),
            pltpu.VMEM((_B, _C), jnp.float32),
            pltpu.VMEM((3 * _C + 8, _BB * _L), jnp.float32),
        ],
    )(x, w104, pool_mat,
      bn_w.reshape(1, -1), bn_b.reshape(1, -1),
      bn_mean.reshape(1, -1), bn_var.reshape(1, -1),
      fc1_w, fc1_b.reshape(1, -1), fc2_w, fc2_b.reshape(1, -1),
      gumbel, expw2, expb2)
    return out


# speedup vs baseline: 1.1444x; 1.1444x over previous
import jax
import jax.numpy as jnp
from jax.experimental import pallas as pl
from jax.experimental.pallas import tpu as pltpu

_B, _C, _L, _E, _D = 64, 32, 2048, 8, 768
_TAU = 1.0
_BB = 16
_NG = _B // _BB


def _gelu(v):
    return v * (0.5 + 0.5 * jax.lax.erf(v * 0.7071067811865476))


def _body(x_ref, w_ref, b_ref, p_ref, bnw_ref, bnb_ref, bnm_ref, bnv_ref,
          fc1w_ref, fc1b_ref, fc2w_ref, fc2b_ref, gum_ref, expw_ref,
          expb_ref, out_ref, feat_s, pooled_s, xs_s):
    i = pl.program_id(0)
    W = w_ref[...]
    b = b_ref[...]
    P = p_ref[...]
    for j in range(_BB):
        X = x_ref[j]
        z = jnp.zeros((_C, 1), jnp.float32)
        Xl = jnp.concatenate([z, X[:, :-1]], axis=1)
        Xr = jnp.concatenate([X[:, 1:], z], axis=1)
        xs_s[:, pl.ds(j * _L, _L)] = jnp.concatenate([Xl, X, Xr], axis=0)
    H = jax.lax.dot_general(W, xs_s[...], (((1,), (0,)), ((), ())),
                            preferred_element_type=jnp.float32)
    H = _gelu(H + b)
    for j in range(_BB):
        feat_s[i * _BB + j] = jnp.dot(H[:, j * _L:(j + 1) * _L], P,
                                      preferred_element_type=jnp.float32)
    pooled_s[pl.ds(i * _BB, _BB)] = jnp.mean(x_ref[...], axis=2)

    @pl.when(i == _NG - 1)
    def _stage2():
        f = feat_s[...].reshape(_B, 64 * 16)
        f = (f - bnm_ref[...]) * jax.lax.rsqrt(bnv_ref[...] + 1e-5) \
            * bnw_ref[...] + bnb_ref[...]
        h1 = _gelu(jnp.dot(f, fc1w_ref[...],
                           preferred_element_type=jnp.float32)
                   + fc1b_ref[...])
        logits = jnp.dot(h1, fc2w_ref[...],
                         preferred_element_type=jnp.float32) + fc2b_ref[...]
        zz = (logits + gum_ref[...]) / _TAU
        zz = zz - jnp.max(zz, axis=1, keepdims=True)
        ez = jnp.exp(zz)
        r = ez / jnp.sum(ez, axis=1, keepdims=True)

        col = jax.lax.broadcasted_iota(jnp.int32, (_B, _E), 1)
        m1 = jnp.max(r, axis=1, keepdims=True)
        i1 = jnp.min(jnp.where(r == m1, col, _E), axis=1, keepdims=True)
        rm = jnp.where(col == i1, -jnp.inf, r)
        m2 = jnp.max(rm, axis=1, keepdims=True)
        i2 = jnp.min(jnp.where(rm == m2, col, _E), axis=1, keepdims=True)
        s = m1 + m2 + 1e-8
        wfull = (jnp.where(col == i1, m1 / s, 0.0)
                 + jnp.where(col == i2, m2 / s, 0.0))

        ao = jnp.tanh(jnp.dot(pooled_s[...], expw_ref[...],
                              preferred_element_type=jnp.float32)
                      + expb_ref[...])
        acc = jnp.zeros((_B, _D), jnp.float32)
        for e in range(_E):
            acc = acc + wfull[:, e:e + 1] * ao[:, e * _D:(e + 1) * _D]
        out_ref[...] = acc


def kernel(x, conv_w, conv_b, bn_w, bn_b, bn_mean, bn_var,
           fc1_w, fc1_b, fc2_w, fc2_b, gumbel, exp_w, exp_b):
    w96 = jnp.transpose(conv_w, (0, 2, 1)).reshape(64, 96)
    cb = conv_b.reshape(64, 1)
    expw2 = jnp.transpose(exp_w, (1, 0, 2)).reshape(_C, _E * _D)
    expb2 = exp_b.reshape(1, _E * _D)
    pool_mat = (jnp.arange(_L)[:, None] // 128
                == jnp.arange(16)[None, :]).astype(jnp.float32) / 128.0

    cst = lambda *dims: pl.BlockSpec(dims, lambda i: (0,) * len(dims))
    out = pl.pallas_call(
        _body,
        grid=(_NG,),
        in_specs=[
            pl.BlockSpec((_BB, _C, _L), lambda i: (i, 0, 0)),
            cst(64, 96), cst(64, 1), cst(_L, 16),
            cst(1, 1024), cst(1, 1024), cst(1, 1024), cst(1, 1024),
            cst(1024, 128), cst(1, 128), cst(128, _E), cst(1, _E),
            cst(_B, _E), cst(_C, _E * _D), cst(1, _E * _D),
        ],
        out_specs=pl.BlockSpec((_B, _D), lambda i: (0, 0)),
        out_shape=jax.ShapeDtypeStruct((_B, _D), jnp.float32),
        scratch_shapes=[
            pltpu.VMEM((_B, 64, 16), jnp.float32),
            pltpu.VMEM((_B, _C), jnp.float32),
            pltpu.VMEM((3 * _C, _BB * _L), jnp.float32),
        ],
    )(x, w96, cb, pool_mat,
      bn_w.reshape(1, -1), bn_b.reshape(1, -1),
      bn_mean.reshape(1, -1), bn_var.reshape(1, -1),
      fc1_w, fc1_b.reshape(1, -1), fc2_w, fc2_b.reshape(1, -1),
      gumbel, expw2, expb2)
    return out
